# Initial kernel scaffold; baseline (speedup 1.0000x reference)
import jax
import jax.numpy as jnp
from jax.experimental import pallas as pl


def kernel(n, table):
    raise NotImplementedError("write your pallas kernel here")


# speedup vs baseline: 39.1839x; 39.1839x over previous
import functools
import math

import jax
import jax.numpy as jnp
from jax import lax
from jax.experimental import pallas as pl
from jax.experimental.pallas import tpu as pltpu
from jax.experimental.pallas import tpu_sc as plsc

_N = 2048
_HEADS = 16
_NUM_BUCKETS = 32
_MAX_DISTANCE = 128
_W = 4096
_WB = 4112
_LANES = 16
_NSHIFT = 16
_DMA_BATCH = 8


def _diag_buckets(n):
    n_zero = jnp.asarray(n, dtype=jnp.int32) - jnp.asarray(n, dtype=jnp.int32)
    k = jnp.arange(_WB, dtype=jnp.int32) + n_zero
    rel = k - (_N - 1)
    nn = -rel
    num_buckets = _NUM_BUCKETS // 2
    ret = (nn < 0).astype(jnp.int32) * num_buckets
    nn = jnp.abs(nn)
    max_exact = num_buckets // 2
    is_small = nn < max_exact
    n_safe = jnp.maximum(nn, 1)
    val_if_large = max_exact + (
        jnp.log(n_safe.astype(jnp.float32) / max_exact)
        / math.log(_MAX_DISTANCE / max_exact)
        * (num_buckets - max_exact)
    ).astype(jnp.int32)
    val_if_large = jnp.minimum(val_if_large, num_buckets - 1)
    return ret + jnp.where(is_small, nn, val_if_large)


def _sc_body(table_hbm, bucket_hbm, out_hbm, table_v, bucket_v, vals_v, sem):
    h = lax.axis_index("s")
    half = lax.axis_index("c")

    pltpu.sync_copy(table_hbm, table_v)
    pltpu.sync_copy(bucket_hbm, bucket_v)

    lane = lax.iota(jnp.int32, _LANES)
    h_vec = jnp.broadcast_to(h, (_LANES,))

    def fill(m0, carry):
        base_k = m0 * _LANES
        for r in range(_NSHIFT):
            bidx = plsc.load_gather(bucket_v, [lane + (base_k + r)])
            v = plsc.load_gather(table_v, [bidx * _HEADS + h_vec])
            off = pl.multiple_of(r * _W + base_k, _LANES)
            vals_v[pl.ds(off, _LANES)] = v
        return carry

    lax.fori_loop(0, _W // _LANES, fill, 0)

    i0 = half * (_N // 2)

    def rows(g, carry):
        i_base = i0 + g * _DMA_BATCH
        copies = []
        for b in range(_DMA_BATCH):
            i = i_base + b
            st = (_N - 1) - i
            r = jnp.bitwise_and(st, _NSHIFT - 1)
            src_off = pl.multiple_of(r * _W + (st - r), _LANES)
            dst_off = pl.multiple_of((h * _N + i) * _N, _N)
            copies.append(
                pltpu.async_copy(
                    vals_v.at[pl.ds(src_off, _N)],
                    out_hbm.at[pl.ds(dst_off, _N)],
                    sem,
                )
            )
        for cp in copies:
            cp.wait()
        return carry

    lax.fori_loop(0, (_N // 2) // _DMA_BATCH, rows, 0)


def kernel(n, table):
    bucket = _diag_buckets(n)
    mesh = plsc.VectorSubcoreMesh(core_axis_name="c", subcore_axis_name="s")
    call = functools.partial(
        pl.kernel,
        mesh=mesh,
        out_type=jax.ShapeDtypeStruct((_HEADS * _N * _N,), jnp.float32),
        scratch_types=[
            pltpu.VMEM((_NUM_BUCKETS * _HEADS,), jnp.float32),
            pltpu.VMEM((_WB,), jnp.int32),
            pltpu.VMEM((_NSHIFT * _W,), jnp.float32),
            pltpu.SemaphoreType.DMA,
        ],
        compiler_params=pltpu.CompilerParams(
            use_tc_tiling_on_sc=False, needs_layout_passes=False
        ),
    )(_sc_body)
    flat = call(table.reshape(-1), bucket)
    return flat.reshape(_HEADS, _N, _N)


# speedup vs baseline: 41.3118x; 1.0543x over previous
import functools
import math

import jax
import jax.numpy as jnp
from jax import lax
from jax.experimental import pallas as pl
from jax.experimental.pallas import tpu as pltpu
from jax.experimental.pallas import tpu_sc as plsc

_N = 2048
_HEADS = 16
_NUM_BUCKETS = 32
_MAX_DISTANCE = 128
_W = 4096
_WB = 4112
_LANES = 16
_NSHIFT = 8
_DMA_BATCH = 8


def _diag_buckets(n):
    n_zero = jnp.asarray(n, dtype=jnp.int32) - jnp.asarray(n, dtype=jnp.int32)
    k = jnp.arange(_WB, dtype=jnp.int32) + n_zero
    rel = k - (_N - 1)
    nn = -rel
    num_buckets = _NUM_BUCKETS // 2
    ret = (nn < 0).astype(jnp.int32) * num_buckets
    nn = jnp.abs(nn)
    max_exact = num_buckets // 2
    is_small = nn < max_exact
    n_safe = jnp.maximum(nn, 1)
    val_if_large = max_exact + (
        jnp.log(n_safe.astype(jnp.float32) / max_exact)
        / math.log(_MAX_DISTANCE / max_exact)
        * (num_buckets - max_exact)
    ).astype(jnp.int32)
    val_if_large = jnp.minimum(val_if_large, num_buckets - 1)
    return ret + jnp.where(is_small, nn, val_if_large)


def _sc_body(table_hbm, bucket_hbm, out_hbm, table_v, bucket_v, vals_v, sem):
    h = lax.axis_index("s")
    half = lax.axis_index("c")

    pltpu.sync_copy(table_hbm, table_v)
    pltpu.sync_copy(bucket_hbm, bucket_v)

    lane = lax.iota(jnp.int32, _LANES)
    h_vec = jnp.broadcast_to(h, (_LANES,))

    def fill(m0, carry):
        base_k = m0 * _LANES
        for r in range(_NSHIFT):
            bidx = plsc.load_gather(bucket_v, [lane + (base_k + r)])
            v = plsc.load_gather(table_v, [bidx * _HEADS + h_vec])
            off = pl.multiple_of(r * _W + base_k, _LANES)
            vals_v[pl.ds(off, _LANES)] = v
        return carry

    lax.fori_loop(0, _W // _LANES, fill, 0)

    i0 = half * (_N // 2)

    def issue(i_base):
        copies = []
        for b in range(_DMA_BATCH):
            i = i_base + b
            st = (_N - 1) - i
            r = jnp.bitwise_and(st, _NSHIFT - 1)
            src_off = pl.multiple_of(r * _W + (st - r), _NSHIFT)
            dst_off = pl.multiple_of((h * _N + i) * _N, _N)
            copies.append(
                pltpu.async_copy(
                    vals_v.at[pl.ds(src_off, _N)],
                    out_hbm.at[pl.ds(dst_off, _N)],
                    sem,
                )
            )
        return copies

    first = issue(i0)

    def rows(g, carry):
        i_base = i0 + g * _DMA_BATCH
        copies = issue(i_base)
        for cp in copies:
            cp.wait()
        return carry

    lax.fori_loop(1, (_N // 2) // _DMA_BATCH, rows, 0)
    for cp in first:
        cp.wait()


def kernel(n, table):
    bucket = _diag_buckets(n)
    mesh = plsc.VectorSubcoreMesh(core_axis_name="c", subcore_axis_name="s")
    call = functools.partial(
        pl.kernel,
        mesh=mesh,
        out_type=jax.ShapeDtypeStruct((_HEADS * _N * _N,), jnp.float32),
        scratch_types=[
            pltpu.VMEM((_NUM_BUCKETS * _HEADS,), jnp.float32),
            pltpu.VMEM((_WB,), jnp.int32),
            pltpu.VMEM((_NSHIFT * _W,), jnp.float32),
            pltpu.SemaphoreType.DMA,
        ],
        compiler_params=pltpu.CompilerParams(
            use_tc_tiling_on_sc=False, needs_layout_passes=False
        ),
    )(_sc_body)
    flat = call(table.reshape(-1), bucket)
    return flat.reshape(_HEADS, _N, _N)


# speedup vs baseline: 76.1005x; 1.8421x over previous
import functools
import math

import jax
import jax.numpy as jnp
from jax import lax
from jax.experimental import pallas as pl
from jax.experimental.pallas import tpu as pltpu
from jax.experimental.pallas import tpu_sc as plsc

_N = 2048
_HEADS = 16
_NUM_BUCKETS = 32
_MAX_DISTANCE = 128
_T = 4096
_WB = 4224
_LANES = 16


def _diag_buckets(n):
    n_zero = jnp.asarray(n, dtype=jnp.int32) - jnp.asarray(n, dtype=jnp.int32)
    k = jnp.arange(_WB, dtype=jnp.int32) + n_zero
    rel = k - (_N - 1)
    nn = -rel
    num_buckets = _NUM_BUCKETS // 2
    ret = (nn < 0).astype(jnp.int32) * num_buckets
    nn = jnp.abs(nn)
    max_exact = num_buckets // 2
    is_small = nn < max_exact
    n_safe = jnp.maximum(nn, 1)
    val_if_large = max_exact + (
        jnp.log(n_safe.astype(jnp.float32) / max_exact)
        / math.log(_MAX_DISTANCE / max_exact)
        * (num_buckets - max_exact)
    ).astype(jnp.int32)
    val_if_large = jnp.minimum(val_if_large, num_buckets - 1)
    return ret + jnp.where(is_small, nn, val_if_large)


def _sc_body(table_hbm, bucket_hbm, out_hbm, table_v, bucket_v, f_v, sem):
    t = lax.axis_index("s")
    c = lax.axis_index("c")
    h0 = c * 8

    pltpu.sync_copy(table_hbm, table_v)
    pltpu.sync_copy(bucket_hbm, bucket_v)

    lane = lax.iota(jnp.int32, _LANES)
    base_shift = 127 - 8 * t

    def fill(h, buf):
        hv = jnp.broadcast_to(h, (_LANES,))

        def body(m0, carry):
            mbase = m0 * _LANES
            for r in range(8):
                bidx = plsc.load_gather(bucket_v, [lane + (mbase + (base_shift - r))])
                v = plsc.load_gather(table_v, [bidx * _HEADS + hv])
                f_v[buf, r, pl.ds(pl.multiple_of(mbase, _LANES), _LANES)] = v
            return carry

        lax.fori_loop(0, _T // _LANES, body, 0)

    fill(h0, 0)

    def head_loop(hl, carry):
        h = h0 + hl
        buf = jnp.bitwise_and(hl, 1)
        copies = []
        for k in range(16):
            i0 = pl.multiple_of(8 * t + 128 * k, 8)
            m0 = 1920 - 128 * k
            copies.append(
                pltpu.async_copy(
                    f_v.at[buf, pl.ds(0, 8), pl.ds(m0, _N)],
                    out_hbm.at[h, pl.ds(i0, 8), pl.ds(0, _N)],
                    sem,
                )
            )
        fill(jnp.minimum(h + 1, h0 + 7), 1 - buf)
        for cp in copies:
            cp.wait()
        return carry

    lax.fori_loop(0, 8, head_loop, 0)


def kernel(n, table):
    bucket = _diag_buckets(n)
    mesh = plsc.VectorSubcoreMesh(core_axis_name="c", subcore_axis_name="s")
    call = functools.partial(
        pl.kernel,
        mesh=mesh,
        out_type=jax.ShapeDtypeStruct((_HEADS, _N, _N), jnp.float32),
        scratch_types=[
            pltpu.VMEM((_NUM_BUCKETS * _HEADS,), jnp.float32),
            pltpu.VMEM((_WB,), jnp.int32),
            pltpu.VMEM((2, 8, _T), jnp.float32),
            pltpu.SemaphoreType.DMA,
        ],
        compiler_params=pltpu.CompilerParams(needs_layout_passes=False),
    )(_sc_body)
    return call(table.reshape(-1), bucket)


# speedup vs baseline: 81.7089x; 1.0737x over previous
import functools
import math

import jax
import jax.numpy as jnp
from jax import lax
from jax.experimental import pallas as pl
from jax.experimental.pallas import tpu as pltpu
from jax.experimental.pallas import tpu_sc as plsc

_N = 2048
_HEADS = 16
_NUM_BUCKETS = 32
_MAX_DISTANCE = 128
_T = 4096
_WB = 4224
_LANES = 16


def _diag_buckets(n):
    n_zero = jnp.asarray(n, dtype=jnp.int32) - jnp.asarray(n, dtype=jnp.int32)
    k = jnp.arange(_WB, dtype=jnp.int32) + n_zero
    rel = k - (_N - 1)
    nn = -rel
    num_buckets = _NUM_BUCKETS // 2
    ret = (nn < 0).astype(jnp.int32) * num_buckets
    nn = jnp.abs(nn)
    max_exact = num_buckets // 2
    is_small = nn < max_exact
    n_safe = jnp.maximum(nn, 1)
    val_if_large = max_exact + (
        jnp.log(n_safe.astype(jnp.float32) / max_exact)
        / math.log(_MAX_DISTANCE / max_exact)
        * (num_buckets - max_exact)
    ).astype(jnp.int32)
    val_if_large = jnp.minimum(val_if_large, num_buckets - 1)
    return ret + jnp.where(is_small, nn, val_if_large)


def _sc_body(table_hbm, bucket_hbm, out_hbm, table_v, bucket_v, bidx_v, f_v, sem):
    t = lax.axis_index("s")
    c = lax.axis_index("c")
    h0 = c * 8

    pltpu.sync_copy(table_hbm, table_v)
    pltpu.sync_copy(bucket_hbm, bucket_v)

    lane = lax.iota(jnp.int32, _LANES)
    base_shift = 127 - 8 * t

    def prefill(m0, carry):
        mbase = m0 * _LANES
        for r in range(8):
            bidx = plsc.load_gather(bucket_v, [lane + (mbase + (base_shift - r))])
            bidx_v[r, pl.ds(pl.multiple_of(mbase, _LANES), _LANES)] = bidx * _HEADS
        return carry

    lax.fori_loop(0, _T // _LANES, prefill, 0)

    def fill(h, buf):
        hv = jnp.broadcast_to(h, (_LANES,))

        def body(m0, carry):
            mbase = m0 * _LANES
            off = pl.multiple_of(mbase, _LANES)
            for r in range(8):
                bvec = bidx_v[r, pl.ds(off, _LANES)]
                v = plsc.load_gather(table_v, [bvec + hv])
                f_v[buf, r, pl.ds(off, _LANES)] = v
            return carry

        lax.fori_loop(0, _T // _LANES, body, 0)

    fill(h0, 0)

    def head_loop(hl, carry):
        h = h0 + hl
        buf = jnp.bitwise_and(hl, 1)
        copies = []
        for k in range(16):
            i0 = pl.multiple_of(8 * t + 128 * k, 8)
            m0 = 1920 - 128 * k
            copies.append(
                pltpu.async_copy(
                    f_v.at[buf, pl.ds(0, 8), pl.ds(m0, _N)],
                    out_hbm.at[h, pl.ds(i0, 8), pl.ds(0, _N)],
                    sem,
                )
            )
        fill(jnp.minimum(h + 1, h0 + 7), 1 - buf)
        for cp in copies:
            cp.wait()
        return carry

    lax.fori_loop(0, 8, head_loop, 0)


def kernel(n, table):
    bucket = _diag_buckets(n)
    mesh = plsc.VectorSubcoreMesh(core_axis_name="c", subcore_axis_name="s")
    call = functools.partial(
        pl.kernel,
        mesh=mesh,
        out_type=jax.ShapeDtypeStruct((_HEADS, _N, _N), jnp.float32),
        scratch_types=[
            pltpu.VMEM((_NUM_BUCKETS * _HEADS,), jnp.float32),
            pltpu.VMEM((_WB,), jnp.int32),
            pltpu.VMEM((8, _T), jnp.int32),
            pltpu.VMEM((2, 8, _T), jnp.float32),
            pltpu.SemaphoreType.DMA,
        ],
        compiler_params=pltpu.CompilerParams(needs_layout_passes=False),
    )(_sc_body)
    return call(table.reshape(-1), bucket)
